# baseline (speedup 1.0000x reference)
import functools

import jax
import jax.numpy as jnp
import numpy as np
from jax.experimental import pallas as pl
from jax.experimental.pallas import tpu as pltpu
from jax.sharding import Mesh, PartitionSpec as P

_B = 256
_K = 10000
_J = 21
_D = 4
_IN = _J * _D
_KB = 1024
_NB = 10
_KP = _KB * _NB
_NN = 5
_BIG = 1e30


def _acos(x):
    ax = jnp.minimum(jnp.abs(x), 1.0 - 1e-6)
    s = jnp.sqrt(1.0 - ax)
    p = jnp.float32(-0.0187293)
    p = p * ax + jnp.float32(0.0742610)
    p = p * ax - jnp.float32(0.2121144)
    p = p * ax + jnp.float32(1.5707288)
    r = s * p
    return jnp.where(x < 0, jnp.float32(np.pi) - r, r)


def _make_kern(rows):
    def _kern(posej_ref, poseflat_ref, trt_ref, pen_ref, w0_ref, b0_ref,
              w1_ref, b1_ref, w2_ref, b2_ref, w3_ref, b3_ref, mmt_ref,
              out_ref, pn_scr, top5_scr):
        p = posej_ref[...]
        ss = jnp.sum(p * p, axis=2, keepdims=True)
        pn_scr[...] = p * jax.lax.rsqrt(jnp.maximum(ss, 1e-24))

        top5_scr[...] = jnp.full((rows, 128), _BIG, jnp.float32)
        iota_c = jax.lax.broadcasted_iota(jnp.int32, (rows, _KB + 128), 1)

        def kblock(kb, carry):
            def jbody(j, acc):
                t = trt_ref[kb, j]
                pj = pn_scr[j]
                d = (pj[:, 0:1] * t[0:1, :] + pj[:, 1:2] * t[1:2, :]
                     + pj[:, 2:3] * t[2:3, :] + pj[:, 3:4] * t[3:4, :])
                return acc + _acos(d)

            dist = jax.lax.fori_loop(
                0, _J, jbody, jnp.zeros((rows, _KB), jnp.float32))
            dist = dist + pen_ref[kb]

            cand = jnp.concatenate([top5_scr[...], dist], axis=1)
            for i in range(_NN):
                m = jnp.min(cand, axis=1, keepdims=True)
                idx = jnp.where(cand == m, iota_c, _KB + 128)
                first = jnp.min(idx, axis=1, keepdims=True)
                cand = jnp.where(iota_c == first, _BIG, cand)
                top5_scr[:, i:i + 1] = m
            return carry

        jax.lax.fori_loop(0, _NB, kblock, 0)

        x = poseflat_ref[...]
        ssf = jnp.dot(x * x, mmt_ref[...],
                      preferred_element_type=jnp.float32)
        xn = x * jax.lax.rsqrt(jnp.maximum(ssf, 1e-24))
        h = jnp.dot(xn, w0_ref[...], preferred_element_type=jnp.float32)
        h = jnp.maximum(h + b0_ref[...], 0.0)
        h = jnp.dot(h, w1_ref[...], preferred_element_type=jnp.float32)
        h = jnp.maximum(h + b1_ref[...], 0.0)
        h = jnp.dot(h, w2_ref[...], preferred_element_type=jnp.float32)
        h = jnp.maximum(h + b2_ref[...], 0.0)
        pred = jnp.dot(h, w3_ref[...], preferred_element_type=jnp.float32)
        pred = pred + b3_ref[...]

        lane = jax.lax.broadcasted_iota(jnp.int32, (rows, 128), 1)
        t5 = top5_scr[...]
        dv = jnp.sum(jnp.where(lane < _NN, t5, 0.0), axis=1,
                     keepdims=True) * (0.5 / _NN)
        out_ref[...] = jnp.sum(jnp.abs(pred - dv), axis=0, keepdims=True)

    return _kern


@jax.jit
def kernel(pose, train_poses, W0, b0, W1, b1, W2, b2, W3, b3):
    ndev = jax.device_count()
    rows = _B // ndev
    posej = jnp.transpose(pose, (1, 0, 2))
    poseflat = pose.reshape(_B, _IN)
    t = jnp.transpose(train_poses, (1, 2, 0))
    t = jnp.pad(t, ((0, 0), (0, 0), (0, _KP - _K)))
    trt = jnp.transpose(t.reshape(_J, _D, _NB, _KB), (2, 0, 1, 3))
    pen = jnp.where(jnp.arange(_KP, dtype=jnp.int32) < _K, 0.0,
                    _BIG).astype(jnp.float32).reshape(_NB, 1, _KB)
    mmt = jnp.asarray(np.kron(np.eye(_J, dtype=np.float32),
                              np.ones((_D, _D), dtype=np.float32)))
    b0r, b1r, b2r = (b.reshape(1, -1) for b in (b0, b1, b2))
    b3r = b3.reshape(1, 1)

    grid_kern = _make_kern(rows)

    def per_core(posej_s, poseflat_s, *rest):
        partial = pl.pallas_call(
            grid_kern,
            out_shape=jax.ShapeDtypeStruct((1, 1), jnp.float32),
            scratch_shapes=[
                pltpu.VMEM((_J, rows, _D), jnp.float32),
                pltpu.VMEM((rows, 128), jnp.float32),
            ],
        )(posej_s, poseflat_s, *rest)
        return jax.lax.psum(partial, 'x') * (1.0 / _B)

    mesh = Mesh(np.array(jax.devices()), ('x',))
    rep = (P(),) * 11
    out = jax.shard_map(
        per_core, mesh=mesh,
        in_specs=(P(None, 'x', None), P('x', None)) + rep,
        out_specs=P(),
        check_vma=False,
    )(posej, poseflat, trt, pen, W0, b0r, W1, b1r, W2, b2r, W3, b3r, mmt)
    return out[0, 0]


# speedup vs baseline: 2.9194x; 2.9194x over previous
import jax
import jax.numpy as jnp
import numpy as np
from jax.experimental import pallas as pl
from jax.experimental.pallas import tpu as pltpu

_B = 256
_K = 10000
_J = 21
_D = 4
_IN = _J * _D
_KB = 2048
_NB = 5
_KP = _KB * _NB
_NN = 5
_BIG = 1e30


def _acos(x):
    ax = jnp.minimum(jnp.abs(x), 1.0 - 1e-6)
    u = 1.0 - ax
    s = u * jax.lax.rsqrt(u)
    p = jnp.float32(0.046167117)
    p = p * ax - jnp.float32(0.20157937)
    p = p * ax + jnp.float32(1.5701435)
    r = s * p
    return jnp.where(x < 0, jnp.float32(np.pi) - r, r)


def _kern(posej_ref, poseflat_ref, trt_ref, pen_ref, w0_ref, b0_ref,
          w1_ref, b1_ref, w2_ref, b2_ref, w3_ref, b3_ref, mmt_ref,
          out_ref, pn_scr, top5_scr):
    p = posej_ref[...]
    ss = jnp.sum(p * p, axis=2, keepdims=True)
    pn_scr[...] = p * jax.lax.rsqrt(jnp.maximum(ss, 1e-24))

    top5_scr[...] = jnp.full((_B, 128), _BIG, jnp.float32)
    iota_c = jax.lax.broadcasted_iota(jnp.int32, (_B, _KB + 128), 1)

    def kblock(kb, carry):
        def jbody(j, acc):
            t = trt_ref[kb, j]
            pj = pn_scr[j]
            d = (pj[:, 0:1] * t[0:1, :] + pj[:, 1:2] * t[1:2, :]
                 + pj[:, 2:3] * t[2:3, :] + pj[:, 3:4] * t[3:4, :])
            return acc + _acos(d)

        dist = jax.lax.fori_loop(
            0, _J, jbody, jnp.zeros((_B, _KB), jnp.float32))
        dist = dist + pen_ref[kb]

        cand = jnp.concatenate([top5_scr[...], dist], axis=1)
        for i in range(_NN):
            m = jnp.min(cand, axis=1, keepdims=True)
            idx = jnp.where(cand == m, iota_c, _KB + 128)
            first = jnp.min(idx, axis=1, keepdims=True)
            cand = jnp.where(iota_c == first, _BIG, cand)
            top5_scr[:, i:i + 1] = m
        return carry

    jax.lax.fori_loop(0, _NB, kblock, 0)

    x = poseflat_ref[...]
    ssf = jnp.dot(x * x, mmt_ref[...], preferred_element_type=jnp.float32)
    xn = x * jax.lax.rsqrt(jnp.maximum(ssf, 1e-24))
    h = jnp.dot(xn, w0_ref[...], preferred_element_type=jnp.float32)
    h = jnp.maximum(h + b0_ref[...], 0.0)
    h = jnp.dot(h, w1_ref[...], preferred_element_type=jnp.float32)
    h = jnp.maximum(h + b1_ref[...], 0.0)
    h = jnp.dot(h, w2_ref[...], preferred_element_type=jnp.float32)
    h = jnp.maximum(h + b2_ref[...], 0.0)
    pred = jnp.dot(h, w3_ref[...], preferred_element_type=jnp.float32)
    pred = pred + b3_ref[...]

    lane = jax.lax.broadcasted_iota(jnp.int32, (_B, 128), 1)
    t5 = top5_scr[...]
    dv = jnp.sum(jnp.where(lane < _NN, t5, 0.0), axis=1,
                 keepdims=True) * (0.5 / _NN)
    out_ref[...] = jnp.sum(jnp.abs(pred - dv), axis=0,
                           keepdims=True) * (1.0 / _B)


@jax.jit
def kernel(pose, train_poses, W0, b0, W1, b1, W2, b2, W3, b3):
    posej = jnp.transpose(pose, (1, 0, 2))
    poseflat = pose.reshape(_B, _IN)
    t = jnp.transpose(train_poses, (1, 2, 0))
    t = jnp.pad(t, ((0, 0), (0, 0), (0, _KP - _K)))
    trt = jnp.transpose(t.reshape(_J, _D, _NB, _KB), (2, 0, 1, 3))
    pen = jnp.where(jnp.arange(_KP, dtype=jnp.int32) < _K, 0.0,
                    _BIG).astype(jnp.float32).reshape(_NB, 1, _KB)
    mmt = jnp.asarray(np.kron(np.eye(_J, dtype=np.float32),
                              np.ones((_D, _D), dtype=np.float32)))
    out = pl.pallas_call(
        _kern,
        out_shape=jax.ShapeDtypeStruct((1, 1), jnp.float32),
        scratch_shapes=[
            pltpu.VMEM((_J, _B, _D), jnp.float32),
            pltpu.VMEM((_B, 128), jnp.float32),
        ],
    )(posej, poseflat, trt, pen, W0, b0.reshape(1, -1), W1,
      b1.reshape(1, -1), W2, b2.reshape(1, -1), W3, b3.reshape(1, 1), mmt)
    return out[0, 0]


# speedup vs baseline: 4.0225x; 1.3779x over previous
import jax
import jax.numpy as jnp
import numpy as np
from jax.experimental import pallas as pl
from jax.experimental.pallas import tpu as pltpu

_B = 256
_K = 10000
_J = 21
_D = 4
_IN = _J * _D
_KB = 2048
_NB = 5
_KP = _KB * _NB
_NN = 5
_BIG = 1e30


_BF = jnp.bfloat16


def _acos16(x):
    ax = jnp.minimum(jnp.abs(x), _BF(0.99609375))
    u = _BF(1.0) - ax
    s = u * jax.lax.rsqrt(u)
    p = _BF(0.046167117)
    p = p * ax - _BF(0.20157937)
    p = p * ax + _BF(1.5701435)
    r = s * p
    return jnp.where(x < 0, _BF(np.pi) - r, r)


def _kern(posej_ref, poseflat_ref, trt_ref, pen_ref, w0_ref, b0_ref,
          w1_ref, b1_ref, w2_ref, b2_ref, w3_ref, b3_ref, mmt_ref,
          out_ref, pn_scr, top5_scr):
    p = posej_ref[...]
    ss = jnp.sum(p * p, axis=2, keepdims=True)
    pn_scr[...] = (p * jax.lax.rsqrt(jnp.maximum(ss, 1e-24))).astype(_BF)

    top5_scr[...] = jnp.full((_B, 128), _BIG, jnp.float32)
    iota_c = jax.lax.broadcasted_iota(jnp.int32, (_B, _KB + 128), 1)

    def kblock(kb, carry):
        def jbody(j, acc):
            t = trt_ref[kb, j]
            pj = pn_scr[j]
            d = (pj[:, 0:1] * t[0:1, :] + pj[:, 1:2] * t[1:2, :]
                 + pj[:, 2:3] * t[2:3, :] + pj[:, 3:4] * t[3:4, :])
            return acc + _acos16(d).astype(jnp.float32)

        dist = jax.lax.fori_loop(
            0, _J, jbody, jnp.zeros((_B, _KB), jnp.float32))
        dist = dist + pen_ref[kb]

        cand = jnp.concatenate([top5_scr[...], dist], axis=1)
        for i in range(_NN):
            m = jnp.min(cand, axis=1, keepdims=True)
            idx = jnp.where(cand == m, iota_c, _KB + 128)
            first = jnp.min(idx, axis=1, keepdims=True)
            cand = jnp.where(iota_c == first, _BIG, cand)
            top5_scr[:, i:i + 1] = m
        return carry

    jax.lax.fori_loop(0, _NB, kblock, 0)

    x = poseflat_ref[...]
    ssf = jnp.dot(x * x, mmt_ref[...], preferred_element_type=jnp.float32)
    xn = x * jax.lax.rsqrt(jnp.maximum(ssf, 1e-24))
    h = jnp.dot(xn, w0_ref[...], preferred_element_type=jnp.float32)
    h = jnp.maximum(h + b0_ref[...], 0.0)
    h = jnp.dot(h, w1_ref[...], preferred_element_type=jnp.float32)
    h = jnp.maximum(h + b1_ref[...], 0.0)
    h = jnp.dot(h, w2_ref[...], preferred_element_type=jnp.float32)
    h = jnp.maximum(h + b2_ref[...], 0.0)
    pred = jnp.dot(h, w3_ref[...], preferred_element_type=jnp.float32)
    pred = pred + b3_ref[...]

    lane = jax.lax.broadcasted_iota(jnp.int32, (_B, 128), 1)
    t5 = top5_scr[...]
    dv = jnp.sum(jnp.where(lane < _NN, t5, 0.0), axis=1,
                 keepdims=True) * (0.5 / _NN)
    out_ref[...] = jnp.sum(jnp.abs(pred - dv), axis=0,
                           keepdims=True) * (1.0 / _B)


@jax.jit
def kernel(pose, train_poses, W0, b0, W1, b1, W2, b2, W3, b3):
    posej = jnp.transpose(pose, (1, 0, 2))
    poseflat = pose.reshape(_B, _IN)
    t = jnp.transpose(train_poses, (1, 2, 0))
    t = jnp.pad(t, ((0, 0), (0, 0), (0, _KP - _K)))
    trt = jnp.transpose(t.reshape(_J, _D, _NB, _KB),
                        (2, 0, 1, 3)).astype(jnp.bfloat16)
    pen = jnp.where(jnp.arange(_KP, dtype=jnp.int32) < _K, 0.0,
                    _BIG).astype(jnp.float32).reshape(_NB, 1, _KB)
    mmt = jnp.asarray(np.kron(np.eye(_J, dtype=np.float32),
                              np.ones((_D, _D), dtype=np.float32)))
    out = pl.pallas_call(
        _kern,
        out_shape=jax.ShapeDtypeStruct((1, 1), jnp.float32),
        scratch_shapes=[
            pltpu.VMEM((_J, _B, _D), jnp.bfloat16),
            pltpu.VMEM((_B, 128), jnp.float32),
        ],
    )(posej, poseflat, trt, pen, W0, b0.reshape(1, -1), W1,
      b1.reshape(1, -1), W2, b2.reshape(1, -1), W3, b3.reshape(1, 1), mmt)
    return out[0, 0]


# speedup vs baseline: 5.0222x; 1.2485x over previous
import jax
import jax.numpy as jnp
import numpy as np
from jax.experimental import pallas as pl
from jax.experimental.pallas import tpu as pltpu

_B = 256
_K = 10000
_J = 21
_D = 4
_IN = _J * _D
_KB = 2048
_NB = 5
_KP = _KB * _NB
_NN = 5
_BIG = 1e30


_BF = jnp.bfloat16


def _acos16(x):
    ax = jnp.minimum(jnp.abs(x), _BF(0.99609375))
    u = _BF(1.0) - ax
    s = u * jax.lax.rsqrt(u)
    p = _BF(0.046167117)
    p = p * ax - _BF(0.20157937)
    p = p * ax + _BF(1.5701435)
    r = s * p
    return jnp.where(x < 0, _BF(np.pi) - r, r)


def _kern(posej_ref, poseflat_ref, trt_ref, pen_ref, w0_ref, b0_ref,
          w1_ref, b1_ref, w2_ref, b2_ref, w3_ref, b3_ref, mmt_ref,
          out_ref, pn_scr, top5_scr):
    p = posej_ref[...]
    ss = jnp.sum(p * p, axis=2, keepdims=True)
    pn_scr[...] = (p * jax.lax.rsqrt(jnp.maximum(ss, 1e-24))).astype(_BF)

    top5_scr[...] = jnp.full((_B, 128), _BIG, jnp.float32)
    ids = jax.lax.broadcasted_iota(jnp.int32, (_B, _KB), 1) + 128

    def kblock(kb, carry):
        def jbody(jj, acc):
            for c in range(3):
                t = trt_ref[kb, 3 * jj + c]
                pj = pn_scr[3 * jj + c]
                d = (pj[:, 0:1] * t[0:1, :] + pj[:, 1:2] * t[1:2, :]
                     + pj[:, 2:3] * t[2:3, :] + pj[:, 3:4] * t[3:4, :])
                acc = acc + _acos16(d).astype(jnp.float32)
            return acc

        dist = jax.lax.fori_loop(
            0, _J // 3, jbody, jnp.zeros((_B, _KB), jnp.float32))
        dist = dist + pen_ref[kb]

        kd = jax.lax.bitcast_convert_type(dist, jnp.int32)
        kd = jax.lax.bitcast_convert_type((kd & ~0xFFF) | ids,
                                          jnp.float32)
        cand = jnp.concatenate([top5_scr[...], kd], axis=1)
        for i in range(_NN):
            m = jnp.min(cand, axis=1, keepdims=True)
            cand = jnp.where(cand == m, _BIG, cand)
            mi = jax.lax.bitcast_convert_type(m, jnp.int32)
            top5_scr[:, i:i + 1] = jax.lax.bitcast_convert_type(
                (mi & ~0xFFF) | i, jnp.float32)
        return carry

    jax.lax.fori_loop(0, _NB, kblock, 0)

    x = poseflat_ref[...]
    ssf = jnp.dot(x * x, mmt_ref[...], preferred_element_type=jnp.float32)
    xn = x * jax.lax.rsqrt(jnp.maximum(ssf, 1e-24))
    h = jnp.dot(xn, w0_ref[...], preferred_element_type=jnp.float32)
    h = jnp.maximum(h + b0_ref[...], 0.0)
    h = jnp.dot(h, w1_ref[...], preferred_element_type=jnp.float32)
    h = jnp.maximum(h + b1_ref[...], 0.0)
    h = jnp.dot(h, w2_ref[...], preferred_element_type=jnp.float32)
    h = jnp.maximum(h + b2_ref[...], 0.0)
    pred = jnp.dot(h, w3_ref[...], preferred_element_type=jnp.float32)
    pred = pred + b3_ref[...]

    lane = jax.lax.broadcasted_iota(jnp.int32, (_B, 128), 1)
    t5 = top5_scr[...]
    dv = jnp.sum(jnp.where(lane < _NN, t5, 0.0), axis=1,
                 keepdims=True) * (0.5 / _NN)
    out_ref[...] = jnp.sum(jnp.abs(pred - dv), axis=0,
                           keepdims=True) * (1.0 / _B)


@jax.jit
def kernel(pose, train_poses, W0, b0, W1, b1, W2, b2, W3, b3):
    posej = jnp.transpose(pose, (1, 0, 2))
    poseflat = pose.reshape(_B, _IN)
    t = jnp.transpose(train_poses, (1, 2, 0))
    t = jnp.pad(t, ((0, 0), (0, 0), (0, _KP - _K)))
    trt = jnp.transpose(t.reshape(_J, _D, _NB, _KB),
                        (2, 0, 1, 3)).astype(jnp.bfloat16)
    pen = jnp.where(jnp.arange(_KP, dtype=jnp.int32) < _K, 0.0,
                    _BIG).astype(jnp.float32).reshape(_NB, 1, _KB)
    mmt = jnp.asarray(np.kron(np.eye(_J, dtype=np.float32),
                              np.ones((_D, _D), dtype=np.float32)))
    out = pl.pallas_call(
        _kern,
        out_shape=jax.ShapeDtypeStruct((1, 1), jnp.float32),
        scratch_shapes=[
            pltpu.VMEM((_J, _B, _D), jnp.bfloat16),
            pltpu.VMEM((_B, 128), jnp.float32),
        ],
    )(posej, poseflat, trt, pen, W0, b0.reshape(1, -1), W1,
      b1.reshape(1, -1), W2, b2.reshape(1, -1), W3, b3.reshape(1, 1), mmt)
    return out[0, 0]


# speedup vs baseline: 5.2718x; 1.0497x over previous
import jax
import jax.numpy as jnp
import numpy as np
from jax.experimental import pallas as pl
from jax.experimental.pallas import tpu as pltpu

_B = 256
_K = 10000
_J = 21
_D = 4
_IN = _J * _D
_KB = 2048
_NB = 5
_KP = _KB * _NB
_NN = 5
_BIG = 1e30


_BF = jnp.bfloat16


def _acos16(x):
    ax = jnp.minimum(jnp.abs(x), _BF(0.99609375))
    u = _BF(1.0) - ax
    s = u * jax.lax.rsqrt(u)
    p = _BF(0.046167117)
    p = p * ax - _BF(0.20157937)
    p = p * ax + _BF(1.5701435)
    r = s * p
    return jnp.where(x < 0, _BF(np.pi) - r, r)


def _kern(posej_ref, poseflat_ref, trt_ref, pen_ref, w0_ref, b0_ref,
          w1_ref, b1_ref, w2_ref, b2_ref, w3_ref, b3_ref, mmt_ref,
          out_ref, pn_scr, top5_scr):
    p = posej_ref[...]
    ss = jnp.sum(p * p, axis=2, keepdims=True)
    pn_scr[...] = (p * jax.lax.rsqrt(jnp.maximum(ss, 1e-24))).astype(_BF)

    top5_scr[...] = jnp.full((_B, 128), _BIG, jnp.float32)
    ids = jax.lax.broadcasted_iota(jnp.int32, (_B, _KB), 1) + 128

    def kblock(kb, carry):
        def jbody(jj, acc):
            for c in range(7):
                t = trt_ref[kb, 7 * jj + c]
                pj = pn_scr[7 * jj + c]
                d = (pj[:, 0:1] * t[0:1, :] + pj[:, 1:2] * t[1:2, :]
                     + pj[:, 2:3] * t[2:3, :] + pj[:, 3:4] * t[3:4, :])
                acc = acc + _acos16(d).astype(jnp.float32)
            return acc

        dist = jax.lax.fori_loop(
            0, _J // 7, jbody, jnp.zeros((_B, _KB), jnp.float32))
        dist = dist + pen_ref[kb]

        kd = jax.lax.bitcast_convert_type(dist, jnp.int32)
        kd = jax.lax.bitcast_convert_type((kd & ~0xFFF) | ids,
                                          jnp.float32)
        cand = jnp.concatenate([top5_scr[...], kd], axis=1)
        for i in range(_NN):
            m = jnp.min(cand, axis=1, keepdims=True)
            cand = jnp.where(cand == m, _BIG, cand)
            mi = jax.lax.bitcast_convert_type(m, jnp.int32)
            top5_scr[:, i:i + 1] = jax.lax.bitcast_convert_type(
                (mi & ~0xFFF) | i, jnp.float32)
        return carry

    jax.lax.fori_loop(0, _NB, kblock, 0)

    x = poseflat_ref[...]
    ssf = jnp.dot(x * x, mmt_ref[...], preferred_element_type=jnp.float32)
    xn = x * jax.lax.rsqrt(jnp.maximum(ssf, 1e-24))
    h = jnp.dot(xn, w0_ref[...], preferred_element_type=jnp.float32)
    h = jnp.maximum(h + b0_ref[...], 0.0)
    h = jnp.dot(h, w1_ref[...], preferred_element_type=jnp.float32)
    h = jnp.maximum(h + b1_ref[...], 0.0)
    h = jnp.dot(h, w2_ref[...], preferred_element_type=jnp.float32)
    h = jnp.maximum(h + b2_ref[...], 0.0)
    pred = jnp.dot(h, w3_ref[...], preferred_element_type=jnp.float32)
    pred = pred + b3_ref[...]

    lane = jax.lax.broadcasted_iota(jnp.int32, (_B, 128), 1)
    t5 = top5_scr[...]
    dv = jnp.sum(jnp.where(lane < _NN, t5, 0.0), axis=1,
                 keepdims=True) * (0.5 / _NN)
    out_ref[...] = jnp.sum(jnp.abs(pred - dv), axis=0,
                           keepdims=True) * (1.0 / _B)


@jax.jit
def kernel(pose, train_poses, W0, b0, W1, b1, W2, b2, W3, b3):
    posej = jnp.transpose(pose, (1, 0, 2))
    poseflat = pose.reshape(_B, _IN)
    t = jnp.transpose(train_poses, (1, 2, 0))
    t = jnp.pad(t, ((0, 0), (0, 0), (0, _KP - _K)))
    trt = jnp.transpose(t.reshape(_J, _D, _NB, _KB),
                        (2, 0, 1, 3)).astype(jnp.bfloat16)
    pen = jnp.where(jnp.arange(_KP, dtype=jnp.int32) < _K, 0.0,
                    _BIG).astype(jnp.float32).reshape(_NB, 1, _KB)
    mmt = jnp.asarray(np.kron(np.eye(_J, dtype=np.float32),
                              np.ones((_D, _D), dtype=np.float32)))
    out = pl.pallas_call(
        _kern,
        out_shape=jax.ShapeDtypeStruct((1, 1), jnp.float32),
        scratch_shapes=[
            pltpu.VMEM((_J, _B, _D), jnp.bfloat16),
            pltpu.VMEM((_B, 128), jnp.float32),
        ],
    )(posej, poseflat, trt, pen, W0, b0.reshape(1, -1), W1,
      b1.reshape(1, -1), W2, b2.reshape(1, -1), W3, b3.reshape(1, 1), mmt)
    return out[0, 0]


# speedup vs baseline: 5.6123x; 1.0646x over previous
import jax
import jax.numpy as jnp
import numpy as np
from jax.experimental import pallas as pl
from jax.experimental.pallas import tpu as pltpu

_B = 256
_K = 10000
_J = 21
_D = 4
_IN = _J * _D
_KB = 2048
_NB = 5
_KP = _KB * _NB
_NN = 5
_BIG = 1e30


_BF = jnp.bfloat16


def _acos16(x):
    ax = jnp.minimum(jnp.abs(x), _BF(0.99609375))
    u = _BF(1.0) - ax
    s = u * jax.lax.rsqrt(u)
    p = _BF(0.046167117)
    p = p * ax - _BF(0.20157937)
    p = p * ax + _BF(1.5701435)
    r = s * p
    return jnp.where(x < 0, _BF(np.pi) - r, r)


def _kern(posej_ref, poseflat_ref, trt_ref, pen_ref, w0_ref, b0_ref,
          w1_ref, b1_ref, w2_ref, b2_ref, w3_ref, b3_ref, mmt_ref,
          out_ref, pn_scr, top5_scr):
    p = posej_ref[...]
    ss = jnp.sum(p * p, axis=2, keepdims=True)
    pn_scr[...] = (p * jax.lax.rsqrt(jnp.maximum(ss, 1e-24))).astype(_BF)

    top5_scr[...] = jnp.full((_B, 128), _BIG, jnp.float32)
    ids = jax.lax.broadcasted_iota(jnp.int32, (_B, _KB), 1) + 128

    def kblock(kb, carry):
        dist = jnp.zeros((_B, _KB), jnp.float32)
        for j in range(_J):
            t = trt_ref[kb, j]
            pj = pn_scr[j]
            d = (pj[:, 0:1] * t[0:1, :] + pj[:, 1:2] * t[1:2, :]
                 + pj[:, 2:3] * t[2:3, :] + pj[:, 3:4] * t[3:4, :])
            dist = dist + _acos16(d).astype(jnp.float32)
        dist = dist + pen_ref[kb]

        kd = jax.lax.bitcast_convert_type(dist, jnp.int32)
        kd = jax.lax.bitcast_convert_type((kd & ~0xFFF) | ids,
                                          jnp.float32)
        cand = jnp.concatenate([top5_scr[...], kd], axis=1)
        for i in range(_NN):
            m = jnp.min(cand, axis=1, keepdims=True)
            cand = jnp.where(cand == m, _BIG, cand)
            mi = jax.lax.bitcast_convert_type(m, jnp.int32)
            top5_scr[:, i:i + 1] = jax.lax.bitcast_convert_type(
                (mi & ~0xFFF) | i, jnp.float32)
        return carry

    jax.lax.fori_loop(0, _NB, kblock, 0)

    x = poseflat_ref[...]
    ssf = jnp.dot(x * x, mmt_ref[...], preferred_element_type=jnp.float32)
    xn = x * jax.lax.rsqrt(jnp.maximum(ssf, 1e-24))
    h = jnp.dot(xn, w0_ref[...], preferred_element_type=jnp.float32)
    h = jnp.maximum(h + b0_ref[...], 0.0)
    h = jnp.dot(h, w1_ref[...], preferred_element_type=jnp.float32)
    h = jnp.maximum(h + b1_ref[...], 0.0)
    h = jnp.dot(h, w2_ref[...], preferred_element_type=jnp.float32)
    h = jnp.maximum(h + b2_ref[...], 0.0)
    pred = jnp.dot(h, w3_ref[...], preferred_element_type=jnp.float32)
    pred = pred + b3_ref[...]

    lane = jax.lax.broadcasted_iota(jnp.int32, (_B, 128), 1)
    t5 = top5_scr[...]
    dv = jnp.sum(jnp.where(lane < _NN, t5, 0.0), axis=1,
                 keepdims=True) * (0.5 / _NN)
    out_ref[...] = jnp.sum(jnp.abs(pred - dv), axis=0,
                           keepdims=True) * (1.0 / _B)


@jax.jit
def kernel(pose, train_poses, W0, b0, W1, b1, W2, b2, W3, b3):
    posej = jnp.transpose(pose, (1, 0, 2))
    poseflat = pose.reshape(_B, _IN)
    t = jnp.transpose(train_poses, (1, 2, 0))
    t = jnp.pad(t, ((0, 0), (0, 0), (0, _KP - _K)))
    trt = jnp.transpose(t.reshape(_J, _D, _NB, _KB),
                        (2, 0, 1, 3)).astype(jnp.bfloat16)
    pen = jnp.where(jnp.arange(_KP, dtype=jnp.int32) < _K, 0.0,
                    _BIG).astype(jnp.float32).reshape(_NB, 1, _KB)
    mmt = jnp.asarray(np.kron(np.eye(_J, dtype=np.float32),
                              np.ones((_D, _D), dtype=np.float32)))
    out = pl.pallas_call(
        _kern,
        out_shape=jax.ShapeDtypeStruct((1, 1), jnp.float32),
        scratch_shapes=[
            pltpu.VMEM((_J, _B, _D), jnp.bfloat16),
            pltpu.VMEM((_B, 128), jnp.float32),
        ],
    )(posej, poseflat, trt, pen, W0, b0.reshape(1, -1), W1,
      b1.reshape(1, -1), W2, b2.reshape(1, -1), W3, b3.reshape(1, 1), mmt)
    return out[0, 0]
